# Initial kernel scaffold; baseline (speedup 1.0000x reference)
import jax
import jax.numpy as jnp
from jax.experimental import pallas as pl


def kernel(embedding_idx, token_table, pos_table):
    raise NotImplementedError("write your pallas kernel here")


# speedup vs baseline: 1.0544x; 1.0544x over previous
import functools

import jax
import jax.numpy as jnp
from jax import lax
from jax.experimental import pallas as pl
from jax.experimental.pallas import tpu as pltpu
from jax.experimental.pallas import tpu_sc as plsc

_B, _S, _D = 4, 2048, 768
_N = _B * _S
_NW = 32
_RPW = _N // _NW
_CH = 32
_NCH = _RPW // _CH
_LANES = _D // 16


def _emb_body(idx_hbm, table_hbm, pos_hbm, out_hbm, idx_v, rows_v, pos_v, sem):
    nc = 2
    wid = lax.axis_index("s") * nc + lax.axis_index("c")
    base = wid * _RPW
    s0 = lax.rem(base, _S)
    pltpu.sync_copy(idx_hbm.at[pl.ds(base, _RPW)], idx_v)

    def chunk_body(ci, carry):
        off = ci * _CH
        cp = pltpu.async_copy(
            table_hbm.at[idx_v.at[pl.ds(off, _CH)]], rows_v, sem)
        pltpu.sync_copy(pos_hbm.at[pl.ds(s0 + off, _CH)], pos_v)
        cp.wait()

        def row_body(r, c2):
            for c in range(_LANES):
                sl = pl.ds(c * 16, 16)
                rows_v[r, sl] = rows_v[r, sl] + pos_v[r, sl]
            return c2

        lax.fori_loop(0, _CH, row_body, 0)
        pltpu.sync_copy(rows_v, out_hbm.at[pl.ds(base + off, _CH)])
        return carry

    lax.fori_loop(0, _NCH, chunk_body, 0)


@jax.jit
def _emb_lookup(idx_flat, token_table, pos_table):
    mesh = plsc.VectorSubcoreMesh(core_axis_name="c", subcore_axis_name="s")
    return pl.kernel(
        _emb_body,
        mesh=mesh,
        out_type=jax.ShapeDtypeStruct((_N, _D), jnp.float32),
        scratch_types=[
            pltpu.VMEM((_RPW,), jnp.int32),
            pltpu.VMEM((_CH, _D), jnp.float32),
            pltpu.VMEM((_CH, _D), jnp.float32),
            pltpu.SemaphoreType.DMA,
        ],
    )(idx_flat, token_table, pos_table)


def kernel(embedding_idx, token_table, pos_table):
    b, s = embedding_idx.shape
    idx_flat = embedding_idx.reshape(b * s).astype(jnp.int32)
    out = _emb_lookup(idx_flat, token_table, pos_table)
    return out.reshape(b, s, token_table.shape[1])
